# Initial kernel scaffold; baseline (speedup 1.0000x reference)
import functools

import jax
import jax.numpy as jnp
from jax import lax
from jax.experimental import pallas as pl
from jax.experimental.pallas import tpu as pltpu
from jax.experimental.pallas import tpu_sc as plsc

B = 4
S = 2048
D = 1024
NC = 2
NS = 16
NW = NC * NS
S_PER_W = S // NW
CH = 32
NSUB = S_PER_W // CH
NCHUNK = NSUB * B
VECS = CH * D // 16


def _sc_body(ids_hbm, pos_hbm, table_hbm, out_hbm,
             idx_v, pos_v, rows0, rows1,
             idx_sem, pos_sem, gsem0, gsem1, ssem0, ssem1):
    c = lax.axis_index("c")
    s = lax.axis_index("s")
    w = s * NC + c
    s_base = w * S_PER_W

    pltpu.async_copy(ids_hbm.at[:, pl.ds(s_base, S_PER_W)], idx_v,
                     idx_sem).wait()

    rows = (rows0, rows1)
    gsem = (gsem0, gsem1)
    ssem = (ssem0, ssem1)
    gather_h = [None, None]
    store_h = [None, None]

    def start_gather(j):
        sub, b = j // B, j % B
        idx_ref = idx_v.at[b, pl.ds(sub * CH, CH)]
        gather_h[j % 2] = pltpu.async_copy(
            table_hbm.at[idx_ref], rows[j % 2], gsem[j % 2])

    def start_pos_load(sub):
        return pltpu.async_copy(
            pos_hbm.at[pl.ds((s_base + sub * CH) * D, CH * D)],
            pos_v, pos_sem)

    start_gather(0)
    pos_h = start_pos_load(0)

    for j in range(NCHUNK):
        sub, b = j // B, j % B
        if j + 1 < NCHUNK:
            if store_h[(j + 1) % 2] is not None:
                store_h[(j + 1) % 2].wait()
            start_gather(j + 1)
        if j == B:
            pos_h = start_pos_load(1)
        if j % B == 0:
            pos_h.wait()
        gather_h[j % 2].wait()
        rbuf = rows[j % 2]

        @plsc.parallel_loop(0, VECS, unroll=8)
        def add_body(i):
            v = pos_v[pl.ds(i * 16, 16)]
            plsc.addupdate(rbuf.at[i >> 6, pl.ds((i & 63) * 16, 16)], v)

        out_row = b * S + s_base + sub * CH
        store_h[j % 2] = pltpu.async_copy(
            rbuf, out_hbm.at[pl.ds(out_row, CH)], ssem[j % 2])

    store_h[0].wait()
    store_h[1].wait()


@functools.partial(
    pl.kernel,
    out_type=jax.ShapeDtypeStruct((B * S, D), jnp.float32),
    mesh=plsc.VectorSubcoreMesh(core_axis_name="c", subcore_axis_name="s"),
    scratch_types=[
        pltpu.VMEM((B, S_PER_W), jnp.int32),
        pltpu.VMEM((CH * D,), jnp.float32),
        pltpu.VMEM((CH, D), jnp.float32),
        pltpu.VMEM((CH, D), jnp.float32),
        pltpu.SemaphoreType.DMA,
        pltpu.SemaphoreType.DMA,
        pltpu.SemaphoreType.DMA,
        pltpu.SemaphoreType.DMA,
        pltpu.SemaphoreType.DMA,
        pltpu.SemaphoreType.DMA,
    ],
)
def _embed_kernel(ids_hbm, pos_hbm, table_hbm, out_hbm, *scratch):
    _sc_body(ids_hbm, pos_hbm, table_hbm, out_hbm, *scratch)


def kernel(input_ids, src_table, pos_table):
    ids = input_ids.astype(jnp.int32)
    pos_flat = jnp.reshape(pos_table, (S * D,))
    out = _embed_kernel(ids, pos_flat, src_table)
    return jnp.reshape(out, (B, S, D))


# speedup vs baseline: 1.0291x; 1.0291x over previous
import functools

import jax
import jax.numpy as jnp
from jax import lax
from jax.experimental import pallas as pl
from jax.experimental.pallas import tpu as pltpu
from jax.experimental.pallas import tpu_sc as plsc

B = 4
S = 2048
D = 1024
NC = 2
NS = 16
NW = NC * NS
S_PER_W = S // NW
CH = 32
NSUB = S_PER_W // CH
NCHUNK = NSUB * B
VECS = CH * D // 16


def _sc_body(ids_hbm, pos_hbm, table_hbm, out_hbm,
             idx_v, pos_v, rows0, rows1,
             idx_sem, pos_sem, gsem0, gsem1, ssem0, ssem1):
    c = lax.axis_index("c")
    s = lax.axis_index("s")
    w = s * NC + c
    s_base = w * S_PER_W

    idx_hs = [pltpu.async_copy(ids_hbm.at[pl.ds(b * S + s_base, S_PER_W)],
                               idx_v.at[b], idx_sem)
              for b in range(B)]
    for h in idx_hs:
        h.wait()

    rows = (rows0, rows1)
    gsem = (gsem0, gsem1)
    ssem = (ssem0, ssem1)
    gather_h = [None, None]
    store_h = [None, None]

    def start_gather(j):
        sub, b = j // B, j % B
        idx_ref = idx_v.at[b, pl.ds(sub * CH, CH)]
        gather_h[j % 2] = pltpu.async_copy(
            table_hbm.at[idx_ref], rows[j % 2], gsem[j % 2])

    def start_pos_load(sub):
        return pltpu.async_copy(
            pos_hbm.at[pl.ds((s_base + sub * CH) * D, CH * D)],
            pos_v, pos_sem)

    start_gather(0)
    pos_h = start_pos_load(0)

    for j in range(NCHUNK):
        sub, b = j // B, j % B
        if j + 1 < NCHUNK:
            if store_h[(j + 1) % 2] is not None:
                store_h[(j + 1) % 2].wait()
            start_gather(j + 1)
        if j == B:
            pos_h = start_pos_load(1)
        if j % B == 0:
            pos_h.wait()
        gather_h[j % 2].wait()
        rbuf = rows[j % 2]

        @plsc.parallel_loop(0, VECS, unroll=8)
        def add_body(i):
            v = pos_v[pl.ds(i * 16, 16)]
            plsc.addupdate(rbuf.at[i >> 6, pl.ds((i & 63) * 16, 16)], v)

        out_row = b * S + s_base + sub * CH
        store_h[j % 2] = pltpu.async_copy(
            rbuf, out_hbm.at[pl.ds(out_row, CH)], ssem[j % 2])

    store_h[0].wait()
    store_h[1].wait()


@functools.partial(
    pl.kernel,
    out_type=jax.ShapeDtypeStruct((B * S, D), jnp.float32),
    mesh=plsc.VectorSubcoreMesh(core_axis_name="c", subcore_axis_name="s"),
    scratch_types=[
        pltpu.VMEM((B, S_PER_W), jnp.int32),
        pltpu.VMEM((CH * D,), jnp.float32),
        pltpu.VMEM((CH, D), jnp.float32),
        pltpu.VMEM((CH, D), jnp.float32),
        pltpu.SemaphoreType.DMA,
        pltpu.SemaphoreType.DMA,
        pltpu.SemaphoreType.DMA,
        pltpu.SemaphoreType.DMA,
        pltpu.SemaphoreType.DMA,
        pltpu.SemaphoreType.DMA,
    ],
)
def _embed_kernel(ids_hbm, pos_hbm, table_hbm, out_hbm, *scratch):
    _sc_body(ids_hbm, pos_hbm, table_hbm, out_hbm, *scratch)


def kernel(input_ids, src_table, pos_table):
    ids = jnp.reshape(input_ids.astype(jnp.int32), (B * S,))
    pos_flat = jnp.reshape(pos_table, (S * D,))
    out = _embed_kernel(ids, pos_flat, src_table)
    return jnp.reshape(out, (B, S, D))


# speedup vs baseline: 1.2254x; 1.1908x over previous
import functools

import jax
import jax.numpy as jnp
from jax import lax
from jax.experimental import pallas as pl
from jax.experimental.pallas import tpu as pltpu
from jax.experimental.pallas import tpu_sc as plsc

B = 4
S = 2048
D = 1024
NC = 2
NS = 16
NW = NC * NS
S_PER_W = S // NW
CH = 32
NSUB = S_PER_W // CH
NCHUNK = NSUB * B
VECS = CH * D // 16


def _sc_body(ids_hbm, pos_hbm, table_hbm, out_hbm,
             idx_v, pos_v, rows0, rows1,
             idx_sem, pos_sem, gsem0, gsem1, ssem0, ssem1):
    c = lax.axis_index("c")
    s = lax.axis_index("s")
    w = s * NC + c
    s_base = w * S_PER_W

    idx_hs = [pltpu.async_copy(ids_hbm.at[b, pl.ds(s_base, S_PER_W)],
                               idx_v.at[b], idx_sem)
              for b in range(B)]
    for h in idx_hs:
        h.wait()

    rows = (rows0, rows1)
    gsem = (gsem0, gsem1)
    ssem = (ssem0, ssem1)
    gather_h = [None, None]
    store_h = [None, None]

    def start_gather(j):
        sub, b = j // B, j % B
        idx_ref = idx_v.at[b, pl.ds(sub * CH, CH)]
        gather_h[j % 2] = pltpu.async_copy(
            table_hbm.at[idx_ref], rows[j % 2], gsem[j % 2])

    def start_pos_load(sub):
        return pltpu.async_copy(
            pos_hbm.at[pl.ds(s_base + sub * CH, CH)], pos_v, pos_sem)

    start_gather(0)
    pos_h = start_pos_load(0)

    for j in range(NCHUNK):
        sub, b = j // B, j % B
        if j + 1 < NCHUNK:
            if store_h[(j + 1) % 2] is not None:
                store_h[(j + 1) % 2].wait()
            start_gather(j + 1)
        if j == B:
            pos_h = start_pos_load(1)
        if j % B == 0:
            pos_h.wait()
        gather_h[j % 2].wait()
        rbuf = rows[j % 2]

        @plsc.parallel_loop(0, VECS, unroll=8)
        def add_body(i):
            r = i >> 6
            col = (i & 63) * 16
            v = pos_v[r, pl.ds(col, 16)]
            plsc.addupdate(rbuf.at[r, pl.ds(col, 16)], v)

        store_h[j % 2] = pltpu.async_copy(
            rbuf, out_hbm.at[b, pl.ds(s_base + sub * CH, CH)], ssem[j % 2])

    store_h[0].wait()
    store_h[1].wait()


@functools.partial(
    pl.kernel,
    out_type=jax.ShapeDtypeStruct((B, S, D), jnp.float32),
    mesh=plsc.VectorSubcoreMesh(core_axis_name="c", subcore_axis_name="s"),
    scratch_types=[
        pltpu.VMEM((B, S_PER_W), jnp.int32),
        pltpu.VMEM((CH, D), jnp.float32),
        pltpu.VMEM((CH, D), jnp.float32),
        pltpu.VMEM((CH, D), jnp.float32),
        pltpu.SemaphoreType.DMA,
        pltpu.SemaphoreType.DMA,
        pltpu.SemaphoreType.DMA,
        pltpu.SemaphoreType.DMA,
        pltpu.SemaphoreType.DMA,
        pltpu.SemaphoreType.DMA,
    ],
)
def _embed_kernel(ids_hbm, pos_hbm, table_hbm, out_hbm, *scratch):
    _sc_body(ids_hbm, pos_hbm, table_hbm, out_hbm, *scratch)


def kernel(input_ids, src_table, pos_table):
    return _embed_kernel(input_ids.astype(jnp.int32), pos_table, src_table)


# speedup vs baseline: 1.3314x; 1.0865x over previous
import functools

import jax
import jax.numpy as jnp
from jax import lax
from jax.experimental import pallas as pl
from jax.experimental.pallas import tpu as pltpu
from jax.experimental.pallas import tpu_sc as plsc

B = 4
S = 2048
D = 1024
NC = 2
NS = 16
NW = NC * NS
S_PER_W = S // NW
CH = 16
NSUB = S_PER_W // CH
NCHUNK = NSUB * B
NBUF = 5
VECS = CH * D // 16
COLS = D // 16


def _sc_body(ids_hbm, pos_hbm, table_hbm, out_hbm,
             idx_v, pos0, pos1, *rest):
    rows = rest[:NBUF]
    idx_sem, psem0, psem1 = rest[NBUF:NBUF + 3]
    gsem = rest[NBUF + 3:2 * NBUF + 3]
    ssem = rest[2 * NBUF + 3:3 * NBUF + 3]
    pos = (pos0, pos1)
    psem = (psem0, psem1)

    c = lax.axis_index("c")
    s = lax.axis_index("s")
    w = s * NC + c
    s_base = w * S_PER_W

    idx_hs = [pltpu.async_copy(ids_hbm.at[b, pl.ds(s_base, S_PER_W)],
                               idx_v.at[b], idx_sem)
              for b in range(B)]
    for h in idx_hs:
        h.wait()

    gather_h = [None] * NBUF
    store_h = [None] * NBUF
    pos_h = [None, None]

    def start_gather(j):
        sub, b = j // B, j % B
        r = j % NBUF
        idx_ref = idx_v.at[b, pl.ds(sub * CH, CH)]
        gather_h[r] = pltpu.async_copy(
            table_hbm.at[idx_ref], rows[r], gsem[r])

    def start_pos_load(sub):
        pos_h[sub % 2] = pltpu.async_copy(
            pos_hbm.at[pl.ds(s_base + sub * CH, CH)],
            pos[sub % 2], psem[sub % 2])

    start_pos_load(0)
    for j in range(NBUF - 1):
        start_gather(j)

    for j in range(NCHUNK):
        sub, b = j // B, j % B
        if j + NBUF - 1 < NCHUNK:
            r_next = (j + NBUF - 1) % NBUF
            if store_h[r_next] is not None:
                store_h[r_next].wait()
                store_h[r_next] = None
            start_gather(j + NBUF - 1)
        if j % B == 0:
            pos_h[sub % 2].wait()
            if sub + 1 < NSUB:
                start_pos_load(sub + 1)
        gather_h[j % NBUF].wait()
        rbuf = rows[j % NBUF]
        pbuf = pos[sub % 2]

        @plsc.parallel_loop(0, VECS, unroll=8)
        def add_body(i):
            r = i // COLS
            col = (i % COLS) * 16
            v = pbuf[r, pl.ds(col, 16)]
            plsc.addupdate(rbuf.at[r, pl.ds(col, 16)], v)

        store_h[j % NBUF] = pltpu.async_copy(
            rbuf, out_hbm.at[b, pl.ds(s_base + sub * CH, CH)],
            ssem[j % NBUF])

    for r in range(NBUF):
        if store_h[r] is not None:
            store_h[r].wait()


@functools.partial(
    pl.kernel,
    out_type=jax.ShapeDtypeStruct((B, S, D), jnp.float32),
    mesh=plsc.VectorSubcoreMesh(core_axis_name="c", subcore_axis_name="s"),
    scratch_types=(
        [pltpu.VMEM((B, S_PER_W), jnp.int32)]
        + [pltpu.VMEM((CH, D), jnp.float32)] * 2
        + [pltpu.VMEM((CH, D), jnp.float32)] * NBUF
        + [pltpu.SemaphoreType.DMA] * (3 + 2 * NBUF)
    ),
)
def _embed_kernel(ids_hbm, pos_hbm, table_hbm, out_hbm, *scratch):
    _sc_body(ids_hbm, pos_hbm, table_hbm, out_hbm, *scratch)


def kernel(input_ids, src_table, pos_table):
    return _embed_kernel(input_ids.astype(jnp.int32), pos_table, src_table)


# speedup vs baseline: 1.4137x; 1.0618x over previous
import functools

import jax
import jax.numpy as jnp
from jax import lax
from jax.experimental import pallas as pl
from jax.experimental.pallas import tpu as pltpu
from jax.experimental.pallas import tpu_sc as plsc

B = 4
S = 2048
D = 1024
NC = 2
NS = 16
NW = NC * NS
S_PER_W = S // NW
CH = 16
NSUB = S_PER_W // CH
NCHUNK = NSUB * B
NBUF = 5
DEPTH = 3
VECS = CH * D // 16
COLS = D // 16


def _sc_body(ids_hbm, pos_hbm, table_hbm, out_hbm,
             idx_v, pos0, pos1, *rest):
    rows = rest[:NBUF]
    idx_sem, psem0, psem1 = rest[NBUF:NBUF + 3]
    gsem = rest[NBUF + 3:2 * NBUF + 3]
    ssem = rest[2 * NBUF + 3:3 * NBUF + 3]
    pos = (pos0, pos1)
    psem = (psem0, psem1)

    c = lax.axis_index("c")
    s = lax.axis_index("s")
    w = s * NC + c
    s_base = w * S_PER_W

    idx_hs = [pltpu.async_copy(ids_hbm.at[b, pl.ds(s_base, S_PER_W)],
                               idx_v.at[b], idx_sem)
              for b in range(B)]
    for h in idx_hs:
        h.wait()

    gather_h = [None] * NBUF
    store_h = [None] * NBUF
    pos_h = [None, None]

    def start_gather(j):
        sub, b = j // B, j % B
        r = j % NBUF
        idx_ref = idx_v.at[b, pl.ds(sub * CH, CH)]
        gather_h[r] = pltpu.async_copy(
            table_hbm.at[idx_ref], rows[r], gsem[r])

    def start_pos_load(sub):
        pos_h[sub % 2] = pltpu.async_copy(
            pos_hbm.at[pl.ds(s_base + sub * CH, CH)],
            pos[sub % 2], psem[sub % 2])

    start_pos_load(0)
    for j in range(DEPTH):
        start_gather(j)

    for j in range(NCHUNK):
        sub, b = j // B, j % B
        if j + DEPTH < NCHUNK:
            r_next = (j + DEPTH) % NBUF
            if store_h[r_next] is not None:
                store_h[r_next].wait()
                store_h[r_next] = None
            start_gather(j + DEPTH)
        if j % B == 0:
            pos_h[sub % 2].wait()
            if sub + 1 < NSUB:
                start_pos_load(sub + 1)
        gather_h[j % NBUF].wait()
        rbuf = rows[j % NBUF]
        pbuf = pos[sub % 2]

        @plsc.parallel_loop(0, VECS, unroll=8)
        def add_body(i):
            r = i // COLS
            col = (i % COLS) * 16
            v = pbuf[r, pl.ds(col, 16)]
            plsc.addupdate(rbuf.at[r, pl.ds(col, 16)], v)

        store_h[j % NBUF] = pltpu.async_copy(
            rbuf, out_hbm.at[b, pl.ds(s_base + sub * CH, CH)],
            ssem[j % NBUF])

    for r in range(NBUF):
        if store_h[r] is not None:
            store_h[r].wait()


@functools.partial(
    pl.kernel,
    out_type=jax.ShapeDtypeStruct((B, S, D), jnp.float32),
    mesh=plsc.VectorSubcoreMesh(core_axis_name="c", subcore_axis_name="s"),
    scratch_types=(
        [pltpu.VMEM((B, S_PER_W), jnp.int32)]
        + [pltpu.VMEM((CH, D), jnp.float32)] * 2
        + [pltpu.VMEM((CH, D), jnp.float32)] * NBUF
        + [pltpu.SemaphoreType.DMA] * (3 + 2 * NBUF)
    ),
)
def _embed_kernel(ids_hbm, pos_hbm, table_hbm, out_hbm, *scratch):
    _sc_body(ids_hbm, pos_hbm, table_hbm, out_hbm, *scratch)


def kernel(input_ids, src_table, pos_table):
    return _embed_kernel(input_ids.astype(jnp.int32), pos_table, src_table)
